# Initial kernel scaffold; baseline (speedup 1.0000x reference)
import jax
import jax.numpy as jnp
from jax.experimental import pallas as pl


def kernel(x, edge_index, W1, b1, W2, b2):
    raise NotImplementedError("write your pallas kernel here")


# speedup vs baseline: 53.8497x; 53.8497x over previous
import functools

import jax
import jax.numpy as jnp
from jax import lax
from jax.experimental import pallas as pl
from jax.experimental.pallas import tpu as pltpu
from jax.experimental.pallas import tpu_sc as plsc

N_NODES = 10000
N_EDGES = 320000
D_IN = 128
D_HID = 16
D_OUT = 128

NC = 2
NS = 16
NW = NC * NS
CHUNK = 128
NCHUNK = 80
EPAD = NW * NCHUNK * CHUNK
NPAD = 10240
RPS = NPAD // NS

_mesh = plsc.VectorSubcoreMesh(
    core_axis_name="c", subcore_axis_name="s", num_cores=NC, num_subcores=NS)


def _fill(ref, n_rows, value):
    val = jnp.full((D_HID,), value, jnp.float32)

    def body(i, carry):
        ref[i] = val
        return carry

    lax.fori_loop(0, n_rows, body, 0)


@functools.partial(
    pl.kernel,
    out_type=jax.ShapeDtypeStruct((NC * NPAD, D_HID), jnp.float32),
    mesh=_mesh,
    scratch_types=[
        pltpu.VMEM((NCHUNK, CHUNK), jnp.int32),
        pltpu.VMEM((CHUNK, D_HID), jnp.float32),
        pltpu.VMEM((RPS, D_HID), jnp.float32),
        pltpu.VMEM_SHARED((NPAD, D_HID), jnp.float32),
    ],
    compiler_params=pltpu.CompilerParams(use_tc_tiling_on_sc=False),
)
def _sc_degree(col_hbm, out_hbm, col_v, ones_v, zeros_v, acc_sh):
    cid = lax.axis_index("c")
    sid = lax.axis_index("s")
    wid = sid * NC + cid
    pltpu.sync_copy(col_hbm.at[wid], col_v)
    _fill(ones_v, CHUNK, 1.0)
    _fill(zeros_v, RPS, 0.0)
    pltpu.sync_copy(zeros_v, acc_sh.at[pl.ds(sid * RPS, RPS)])
    plsc.subcore_barrier()

    def body(j, carry):
        pltpu.sync_copy(ones_v, acc_sh.at[col_v.at[j]], add=True)
        return carry

    lax.fori_loop(0, NCHUNK, body, 0)
    plsc.subcore_barrier()
    pltpu.sync_copy(acc_sh.at[pl.ds(sid * RPS, RPS)],
                    out_hbm.at[pl.ds(cid * NPAD + sid * RPS, RPS)])


@functools.partial(
    pl.kernel,
    out_type=jax.ShapeDtypeStruct((NC * NPAD, D_HID), jnp.float32),
    mesh=_mesh,
    scratch_types=[
        pltpu.VMEM((NCHUNK, CHUNK), jnp.int32),
        pltpu.VMEM((NCHUNK, CHUNK), jnp.int32),
        pltpu.VMEM((CHUNK, D_HID), jnp.float32),
        pltpu.VMEM((CHUNK, D_HID), jnp.float32),
        pltpu.VMEM((RPS, D_HID), jnp.float32),
        pltpu.VMEM_SHARED((NPAD, D_HID), jnp.float32),
        pltpu.VMEM_SHARED((NPAD, D_HID), jnp.float32),
        pltpu.SemaphoreType.DMA,
        pltpu.SemaphoreType.DMA,
    ],
    compiler_params=pltpu.CompilerParams(use_tc_tiling_on_sc=False),
)
def _sc_aggregate(ht_hbm, row_hbm, col_hbm, out_hbm,
                  row_v, col_v, msg0_v, msg1_v, zeros_v,
                  acc_sh, ht_sh, sem0, sem1):
    cid = lax.axis_index("c")
    sid = lax.axis_index("s")
    wid = sid * NC + cid
    pltpu.sync_copy(ht_hbm.at[pl.ds(sid * RPS, RPS)],
                    ht_sh.at[pl.ds(sid * RPS, RPS)])
    pltpu.sync_copy(row_hbm.at[wid], row_v)
    pltpu.sync_copy(col_hbm.at[wid], col_v)
    _fill(zeros_v, RPS, 0.0)
    pltpu.sync_copy(zeros_v, acc_sh.at[pl.ds(sid * RPS, RPS)])
    plsc.subcore_barrier()

    def body(t, carry):
        j0 = 2 * t
        j1 = 2 * t + 1
        d0 = pltpu.async_copy(ht_sh.at[row_v.at[j0]], msg0_v, sem0)
        d1 = pltpu.async_copy(ht_sh.at[row_v.at[j1]], msg1_v, sem1)
        d0.wait()
        pltpu.sync_copy(msg0_v, acc_sh.at[col_v.at[j0]], add=True)
        d1.wait()
        pltpu.sync_copy(msg1_v, acc_sh.at[col_v.at[j1]], add=True)
        return carry

    lax.fori_loop(0, NCHUNK // 2, body, 0)
    plsc.subcore_barrier()
    pltpu.sync_copy(acc_sh.at[pl.ds(sid * RPS, RPS)],
                    out_hbm.at[pl.ds(cid * NPAD + sid * RPS, RPS)])


def _tc_pre(deg_ref, x_ref, w1_ref, dinv_ref, ht1_ref):
    deg = deg_ref[0] + deg_ref[1] + 1.0
    dinv = lax.rsqrt(deg)
    dinv_ref[...] = dinv
    xw = jnp.dot(x_ref[...], w1_ref[...], preferred_element_type=jnp.float32)
    ht1_ref[0:N_NODES] = dinv[0:N_NODES] * xw
    ht1_ref[N_NODES:NPAD] = jnp.zeros((NPAD - N_NODES, D_HID), jnp.float32)


def _tc_mid(acc_ref, ht1_ref, dinv_ref, b1_ref, ht2_ref):
    dinv = dinv_ref[...]
    s = acc_ref[0] + acc_ref[1] + ht1_ref[...]
    out1 = jnp.maximum(dinv * s + b1_ref[...], 0.0)
    ht2_ref[...] = dinv * out1


def _tc_final(acc_ref, ht2_ref, dinv_ref, w2_ref, b2_ref, out_ref):
    g = dinv_ref[...] * (acc_ref[0] + acc_ref[1] + ht2_ref[...])
    h = jnp.dot(g, w2_ref[...], preferred_element_type=jnp.float32)
    h = h + b2_ref[...]
    m = jnp.max(h, axis=1, keepdims=True)
    lse = m + jnp.log(jnp.sum(jnp.exp(h - m), axis=1, keepdims=True))
    out_ref[...] = h - lse


def kernel(x, edge_index, W1, b1, W2, b2):
    ei = edge_index.astype(jnp.int32)
    row = jnp.concatenate(
        [ei[0], jnp.zeros((EPAD - N_EDGES,), jnp.int32)]).reshape(
            NW, NCHUNK, CHUNK)
    pad_col = N_NODES + jnp.arange(EPAD - N_EDGES, dtype=jnp.int32) % (
        NPAD - N_NODES)
    col = jnp.concatenate([ei[1], pad_col]).reshape(NW, NCHUNK, CHUNK)

    deg2 = _sc_degree(col).reshape(NC, NPAD, D_HID)

    dinv, ht1 = pl.pallas_call(
        _tc_pre,
        out_shape=[
            jax.ShapeDtypeStruct((NPAD, D_HID), jnp.float32),
            jax.ShapeDtypeStruct((NPAD, D_HID), jnp.float32),
        ],
    )(deg2, x, W1)

    acc1 = _sc_aggregate(ht1, row, col).reshape(NC, NPAD, D_HID)

    ht2 = pl.pallas_call(
        _tc_mid,
        out_shape=jax.ShapeDtypeStruct((NPAD, D_HID), jnp.float32),
    )(acc1, ht1, dinv, b1.reshape(1, D_HID))

    acc2 = _sc_aggregate(ht2, row, col).reshape(NC, NPAD, D_HID)

    out = pl.pallas_call(
        _tc_final,
        out_shape=jax.ShapeDtypeStruct((NPAD, D_OUT), jnp.float32),
    )(acc2, ht2, dinv, W2, b2.reshape(1, D_OUT))

    return out[:N_NODES]


# speedup vs baseline: 54.8925x; 1.0194x over previous
import functools

import jax
import jax.numpy as jnp
from jax import lax
from jax.experimental import pallas as pl
from jax.experimental.pallas import tpu as pltpu
from jax.experimental.pallas import tpu_sc as plsc

N_NODES = 10000
N_EDGES = 320000
D_IN = 128
D_HID = 16
D_OUT = 128

NC = 2
NS = 16
NW = NC * NS
CHUNK = 128
NCHUNK = 80
EPAD = NW * NCHUNK * CHUNK
NPAD = 10240
RPS = NPAD // NS

_mesh = plsc.VectorSubcoreMesh(
    core_axis_name="c", subcore_axis_name="s", num_cores=NC, num_subcores=NS)


def _fill(ref, n_rows, value):
    val = jnp.full((D_HID,), value, jnp.float32)

    def body(i, carry):
        ref[i] = val
        return carry

    lax.fori_loop(0, n_rows, body, 0)


@functools.partial(
    pl.kernel,
    out_type=jax.ShapeDtypeStruct((NC * NPAD, D_HID), jnp.float32),
    mesh=_mesh,
    scratch_types=[
        pltpu.VMEM((NCHUNK, CHUNK), jnp.int32),
        pltpu.VMEM((CHUNK, D_HID), jnp.float32),
        pltpu.VMEM((RPS, D_HID), jnp.float32),
        pltpu.VMEM_SHARED((NPAD, D_HID), jnp.float32),
    ],
    compiler_params=pltpu.CompilerParams(use_tc_tiling_on_sc=False),
)
def _sc_degree(col_hbm, out_hbm, col_v, ones_v, zeros_v, acc_sh):
    cid = lax.axis_index("c")
    sid = lax.axis_index("s")
    wid = sid * NC + cid
    pltpu.sync_copy(col_hbm.at[wid], col_v)
    _fill(ones_v, CHUNK, 1.0)
    _fill(zeros_v, RPS, 0.0)
    pltpu.sync_copy(zeros_v, acc_sh.at[pl.ds(sid * RPS, RPS)])
    plsc.subcore_barrier()

    def body(j, carry):
        pltpu.sync_copy(ones_v, acc_sh.at[col_v.at[j]], add=True)
        return carry

    lax.fori_loop(0, NCHUNK, body, 0)
    plsc.subcore_barrier()
    pltpu.sync_copy(acc_sh.at[pl.ds(sid * RPS, RPS)],
                    out_hbm.at[pl.ds(cid * NPAD + sid * RPS, RPS)])


@functools.partial(
    pl.kernel,
    out_type=jax.ShapeDtypeStruct((NC * NPAD, D_HID), jnp.float32),
    mesh=_mesh,
    scratch_types=[
        pltpu.VMEM((NCHUNK, CHUNK), jnp.int32),
        pltpu.VMEM((NCHUNK, CHUNK), jnp.int32),
        pltpu.VMEM((CHUNK, D_HID), jnp.float32),
        pltpu.VMEM((CHUNK, D_HID), jnp.float32),
        pltpu.VMEM((RPS, D_HID), jnp.float32),
        pltpu.VMEM_SHARED((NPAD, D_HID), jnp.float32),
        pltpu.VMEM_SHARED((NPAD, D_HID), jnp.float32),
        pltpu.SemaphoreType.DMA,
        pltpu.SemaphoreType.DMA,
    ],
    compiler_params=pltpu.CompilerParams(use_tc_tiling_on_sc=False),
)
def _sc_aggregate(ht_hbm, row_hbm, col_hbm, out_hbm,
                  row_v, col_v, msg0_v, msg1_v, zeros_v,
                  acc_sh, ht_sh, sem0, sem1):
    cid = lax.axis_index("c")
    sid = lax.axis_index("s")
    wid = sid * NC + cid
    pltpu.sync_copy(ht_hbm.at[pl.ds(sid * RPS, RPS)],
                    ht_sh.at[pl.ds(sid * RPS, RPS)])
    pltpu.sync_copy(row_hbm.at[wid], row_v)
    pltpu.sync_copy(col_hbm.at[wid], col_v)
    _fill(zeros_v, RPS, 0.0)
    pltpu.sync_copy(zeros_v, acc_sh.at[pl.ds(sid * RPS, RPS)])
    plsc.subcore_barrier()

    def body(t, carry):
        j0 = 2 * t
        j1 = 2 * t + 1
        d0 = pltpu.async_copy(ht_sh.at[row_v.at[j0]], msg0_v, sem0)
        d1 = pltpu.async_copy(ht_sh.at[row_v.at[j1]], msg1_v, sem1)
        d0.wait()
        pltpu.sync_copy(msg0_v, acc_sh.at[col_v.at[j0]], add=True)
        d1.wait()
        pltpu.sync_copy(msg1_v, acc_sh.at[col_v.at[j1]], add=True)
        return carry

    lax.fori_loop(0, NCHUNK // 2, body, 0)
    plsc.subcore_barrier()
    pltpu.sync_copy(acc_sh.at[pl.ds(sid * RPS, RPS)],
                    out_hbm.at[pl.ds(cid * NPAD + sid * RPS, RPS)])


def _tc_pre(deg_ref, x_ref, w1_ref, dinv_ref, ht1_ref):
    deg = deg_ref[0:NPAD] + deg_ref[NPAD:NC * NPAD] + 1.0
    dinv = lax.rsqrt(deg)
    dinv_ref[...] = dinv
    xw = jnp.dot(x_ref[...], w1_ref[...], preferred_element_type=jnp.float32)
    ht1_ref[0:N_NODES] = dinv[0:N_NODES] * xw
    ht1_ref[N_NODES:NPAD] = jnp.zeros((NPAD - N_NODES, D_HID), jnp.float32)


def _tc_mid(acc_ref, ht1_ref, dinv_ref, b1_ref, ht2_ref):
    dinv = dinv_ref[...]
    s = acc_ref[0:NPAD] + acc_ref[NPAD:NC * NPAD] + ht1_ref[...]
    out1 = jnp.maximum(dinv * s + b1_ref[...], 0.0)
    ht2_ref[...] = dinv * out1


def _tc_final(acc_ref, ht2_ref, dinv_ref, w2_ref, b2_ref, out_ref):
    g = dinv_ref[0:N_NODES] * (
        acc_ref[0:N_NODES] + acc_ref[NPAD:NPAD + N_NODES] +
        ht2_ref[0:N_NODES])
    h = jnp.dot(g, w2_ref[...], preferred_element_type=jnp.float32)
    h = h + b2_ref[...]
    m = jnp.max(h, axis=1, keepdims=True)
    lse = m + jnp.log(jnp.sum(jnp.exp(h - m), axis=1, keepdims=True))
    out_ref[...] = h - lse


def kernel(x, edge_index, W1, b1, W2, b2):
    ei = edge_index.astype(jnp.int32)
    row = jnp.concatenate(
        [ei[0], jnp.zeros((EPAD - N_EDGES,), jnp.int32)]).reshape(
            NW, NCHUNK, CHUNK)
    pad_col = N_NODES + jnp.arange(EPAD - N_EDGES, dtype=jnp.int32) % (
        NPAD - N_NODES)
    col = jnp.concatenate([ei[1], pad_col]).reshape(NW, NCHUNK, CHUNK)

    deg2 = _sc_degree(col)

    dinv, ht1 = pl.pallas_call(
        _tc_pre,
        out_shape=[
            jax.ShapeDtypeStruct((NPAD, D_HID), jnp.float32),
            jax.ShapeDtypeStruct((NPAD, D_HID), jnp.float32),
        ],
    )(deg2, x, W1)

    acc1 = _sc_aggregate(ht1, row, col)

    ht2 = pl.pallas_call(
        _tc_mid,
        out_shape=jax.ShapeDtypeStruct((NPAD, D_HID), jnp.float32),
    )(acc1, ht1, dinv, b1.reshape(1, D_HID))

    acc2 = _sc_aggregate(ht2, row, col)

    out = pl.pallas_call(
        _tc_final,
        out_shape=jax.ShapeDtypeStruct((N_NODES, D_OUT), jnp.float32),
    )(acc2, ht2, dinv, W2, b2.reshape(1, D_OUT))

    return out


# speedup vs baseline: 72.2653x; 1.3165x over previous
import functools

import jax
import jax.numpy as jnp
import numpy as np
from jax import lax
from jax.experimental import pallas as pl
from jax.experimental.pallas import tpu as pltpu
from jax.experimental.pallas import tpu_sc as plsc

N_NODES = 10000
N_EDGES = 320000
D_IN = 128
D_HID = 16
D_OUT = 128

NC = 2
NS = 16
NW = NC * NS
CHUNK = 128
NCHUNK = 80
EPAD = NW * NCHUNK * CHUNK
NPAD = 10240
RPS = NPAD // NS
NP8 = NPAD // 8
NN8 = N_NODES // 8

_PAD_ROW = np.zeros(EPAD - N_EDGES, np.int32)
_PAD_COL = (N_NODES + np.arange(EPAD - N_EDGES) % (NPAD - N_NODES)).astype(
    np.int32)

_mesh = plsc.VectorSubcoreMesh(
    core_axis_name="c", subcore_axis_name="s", num_cores=NC, num_subcores=NS)


def _fill(ref, n_rows, value):
    val = jnp.full((D_HID,), value, jnp.float32)

    def body(i, carry):
        ref[i] = val
        return carry

    lax.fori_loop(0, n_rows, body, 0)


@functools.partial(
    pl.kernel,
    out_type=jax.ShapeDtypeStruct((NC * NPAD, D_HID), jnp.float32),
    mesh=_mesh,
    scratch_types=[
        pltpu.VMEM((NCHUNK, CHUNK), jnp.int32),
        pltpu.VMEM((CHUNK, D_HID), jnp.float32),
        pltpu.VMEM((RPS, D_HID), jnp.float32),
        pltpu.VMEM_SHARED((NPAD, D_HID), jnp.float32),
    ],
    compiler_params=pltpu.CompilerParams(use_tc_tiling_on_sc=False),
)
def _sc_degree(col_hbm, out_hbm, col_v, ones_v, zeros_v, acc_sh):
    cid = lax.axis_index("c")
    sid = lax.axis_index("s")
    wid = sid * NC + cid
    pltpu.sync_copy(col_hbm.at[pl.ds(wid * NCHUNK, NCHUNK)], col_v)
    _fill(ones_v, CHUNK, 1.0)
    _fill(zeros_v, RPS, 0.0)
    pltpu.sync_copy(zeros_v, acc_sh.at[pl.ds(sid * RPS, RPS)])
    plsc.subcore_barrier()

    def body(j, carry):
        pltpu.sync_copy(ones_v, acc_sh.at[col_v.at[j]], add=True)
        return carry

    lax.fori_loop(0, NCHUNK, body, 0)
    plsc.subcore_barrier()
    pltpu.sync_copy(acc_sh.at[pl.ds(sid * RPS, RPS)],
                    out_hbm.at[pl.ds(cid * NPAD + sid * RPS, RPS)])


@functools.partial(
    pl.kernel,
    out_type=jax.ShapeDtypeStruct((NC * NPAD, D_HID), jnp.float32),
    mesh=_mesh,
    scratch_types=[
        pltpu.VMEM((NCHUNK, CHUNK), jnp.int32),
        pltpu.VMEM((NCHUNK, CHUNK), jnp.int32),
        pltpu.VMEM((CHUNK, D_HID), jnp.float32),
        pltpu.VMEM((CHUNK, D_HID), jnp.float32),
        pltpu.VMEM((RPS, D_HID), jnp.float32),
        pltpu.VMEM_SHARED((NPAD, D_HID), jnp.float32),
        pltpu.VMEM_SHARED((NPAD, D_HID), jnp.float32),
        pltpu.SemaphoreType.DMA,
        pltpu.SemaphoreType.DMA,
    ],
    compiler_params=pltpu.CompilerParams(use_tc_tiling_on_sc=False),
)
def _sc_aggregate(ht_hbm, row_hbm, col_hbm, out_hbm,
                  row_v, col_v, msg0_v, msg1_v, zeros_v,
                  acc_sh, ht_sh, sem0, sem1):
    cid = lax.axis_index("c")
    sid = lax.axis_index("s")
    wid = sid * NC + cid
    pltpu.sync_copy(ht_hbm.at[pl.ds(sid * RPS, RPS)],
                    ht_sh.at[pl.ds(sid * RPS, RPS)])
    pltpu.sync_copy(row_hbm.at[pl.ds(wid * NCHUNK, NCHUNK)], row_v)
    pltpu.sync_copy(col_hbm.at[pl.ds(wid * NCHUNK, NCHUNK)], col_v)
    _fill(zeros_v, RPS, 0.0)
    pltpu.sync_copy(zeros_v, acc_sh.at[pl.ds(sid * RPS, RPS)])
    plsc.subcore_barrier()

    def body(t, carry):
        j0 = 2 * t
        j1 = 2 * t + 1
        d0 = pltpu.async_copy(ht_sh.at[row_v.at[j0]], msg0_v, sem0)
        d1 = pltpu.async_copy(ht_sh.at[row_v.at[j1]], msg1_v, sem1)
        d0.wait()
        pltpu.sync_copy(msg0_v, acc_sh.at[col_v.at[j0]], add=True)
        d1.wait()
        pltpu.sync_copy(msg1_v, acc_sh.at[col_v.at[j1]], add=True)
        return carry

    lax.fori_loop(0, NCHUNK // 2, body, 0)
    plsc.subcore_barrier()
    pltpu.sync_copy(acc_sh.at[pl.ds(sid * RPS, RPS)],
                    out_hbm.at[pl.ds(cid * NPAD + sid * RPS, RPS)])


def _tc_pre(deg_ref, x3_ref, w1_ref, dinvp_ref, ht1p_ref):
    degp = deg_ref[0:NP8] + deg_ref[NP8:NC * NP8] + 1.0
    dinvp = lax.rsqrt(degp)
    dinvp_ref[...] = dinvp
    xw = [jnp.dot(x3_ref[:, a, :], w1_ref[...],
                  preferred_element_type=jnp.float32) for a in range(8)]
    ht1p_ref[0:NN8] = dinvp[0:NN8] * jnp.concatenate(xw, axis=1)
    ht1p_ref[NN8:NP8] = jnp.zeros((NP8 - NN8, 128), jnp.float32)


def _tc_mid(acc_ref, ht1p_ref, dinvp_ref, b1p_ref, ht2p_ref):
    dinvp = dinvp_ref[...]
    s = acc_ref[0:NP8] + acc_ref[NP8:NC * NP8] + ht1p_ref[...]
    out1 = jnp.maximum(dinvp * s + b1p_ref[...], 0.0)
    ht2p_ref[...] = dinvp * out1


def _tc_final(acc_ref, ht2p_ref, dinvp_ref, w2_ref, b2_ref, out3_ref):
    gp = dinvp_ref[0:NP8] * (
        acc_ref[0:NP8] + acc_ref[NP8:NC * NP8] + ht2p_ref[...])
    for a in range(8):
        ga = gp[0:NN8, 16 * a:16 * (a + 1)]
        h = jnp.dot(ga, w2_ref[...], preferred_element_type=jnp.float32)
        h = h + b2_ref[...]
        m = jnp.max(h, axis=1, keepdims=True)
        lse = m + jnp.log(jnp.sum(jnp.exp(h - m), axis=1, keepdims=True))
        out3_ref[:, a, :] = h - lse


def kernel(x, edge_index, W1, b1, W2, b2):
    ei = edge_index.astype(jnp.int32)
    row = jnp.concatenate([ei[0], jnp.asarray(_PAD_ROW)]).reshape(
        NW * NCHUNK, CHUNK)
    col = jnp.concatenate([ei[1], jnp.asarray(_PAD_COL)]).reshape(
        NW * NCHUNK, CHUNK)
    x3 = x.reshape(NN8, 8, D_IN)

    deg2 = _sc_degree(col)

    dinvp, ht1p = pl.pallas_call(
        _tc_pre,
        out_shape=[
            jax.ShapeDtypeStruct((NP8, 128), jnp.float32),
            jax.ShapeDtypeStruct((NP8, 128), jnp.float32),
        ],
    )(deg2.reshape(NC * NP8, 128), x3, W1)

    acc1 = _sc_aggregate(ht1p.reshape(NPAD, D_HID), row, col)

    ht2p = pl.pallas_call(
        _tc_mid,
        out_shape=jax.ShapeDtypeStruct((NP8, 128), jnp.float32),
    )(acc1.reshape(NC * NP8, 128), ht1p, dinvp,
      jnp.tile(b1.reshape(1, D_HID), (1, 8)))

    acc2 = _sc_aggregate(ht2p.reshape(NPAD, D_HID), row, col)

    out3 = pl.pallas_call(
        _tc_final,
        out_shape=jax.ShapeDtypeStruct((NN8, 8, D_OUT), jnp.float32),
    )(acc2.reshape(NC * NP8, 128), ht2p, dinvp, W2, b2.reshape(1, D_OUT))

    return out3.reshape(N_NODES, D_OUT)


# speedup vs baseline: 86.1409x; 1.1920x over previous
import functools

import jax
import jax.numpy as jnp
import numpy as np
from jax import lax
from jax.experimental import pallas as pl
from jax.experimental.pallas import tpu as pltpu
from jax.experimental.pallas import tpu_sc as plsc

N_NODES = 10000
N_EDGES = 320000
D_IN = 128
D_HID = 16
D_OUT = 128

NC = 2
NS = 16
NW = NC * NS
CHUNK = 128
NCHUNK = 80
EPAD = NW * NCHUNK * CHUNK
NPAD = 10240
RPS = NPAD // NS
NP8 = NPAD // 8
NN8 = N_NODES // 8

_PAD_ROW = np.zeros(EPAD - N_EDGES, np.int32)
_PAD_COL = (N_NODES + np.arange(EPAD - N_EDGES) % (NPAD - N_NODES)).astype(
    np.int32)

_mesh = plsc.VectorSubcoreMesh(
    core_axis_name="c", subcore_axis_name="s", num_cores=NC, num_subcores=NS)


def _fill(ref, n_rows, value):
    val = jnp.full((D_HID,), value, jnp.float32)

    def body(i, carry):
        ref[i] = val
        return carry

    lax.fori_loop(0, n_rows, body, 0)


@functools.partial(
    pl.kernel,
    out_type=jax.ShapeDtypeStruct((NC * NPAD, D_HID), jnp.float32),
    mesh=_mesh,
    scratch_types=[
        pltpu.VMEM((NCHUNK, CHUNK), jnp.int32),
        pltpu.VMEM((CHUNK, D_HID), jnp.float32),
        pltpu.VMEM((CHUNK, D_HID), jnp.float32),
        pltpu.VMEM_SHARED((NPAD, D_HID), jnp.float32),
        pltpu.SemaphoreType.DMA,
        pltpu.SemaphoreType.DMA,
        pltpu.SemaphoreType.DMA,
        pltpu.SemaphoreType.DMA,
    ],
    compiler_params=pltpu.CompilerParams(use_tc_tiling_on_sc=False),
)
def _sc_degree(col_hbm, out_hbm, col_v, ones_v, zeros_v, acc_sh,
               sem0, sem1, sem2, sem3):
    cid = lax.axis_index("c")
    sid = lax.axis_index("s")
    wid = sid * NC + cid
    _fill(ones_v, CHUNK, 1.0)
    _fill(zeros_v, CHUNK, 0.0)
    dc = pltpu.async_copy(col_hbm.at[pl.ds(wid * NCHUNK, NCHUNK)], col_v, sem0)
    dz = [pltpu.async_copy(
        zeros_v, acc_sh.at[pl.ds(sid * RPS + k * CHUNK, CHUNK)], sem1)
        for k in range(RPS // CHUNK)]
    dc.wait()
    for d in dz:
        d.wait()
    plsc.subcore_barrier()

    def body(t, carry):
        s0 = pltpu.async_copy(ones_v, acc_sh.at[col_v.at[4 * t]], sem0,
                              add=True)
        s1 = pltpu.async_copy(ones_v, acc_sh.at[col_v.at[4 * t + 1]], sem1,
                              add=True)
        s2 = pltpu.async_copy(ones_v, acc_sh.at[col_v.at[4 * t + 2]], sem2,
                              add=True)
        s3 = pltpu.async_copy(ones_v, acc_sh.at[col_v.at[4 * t + 3]], sem3,
                              add=True)
        s0.wait()
        s1.wait()
        s2.wait()
        s3.wait()
        return carry

    lax.fori_loop(0, NCHUNK // 4, body, 0)
    plsc.subcore_barrier()
    pltpu.sync_copy(acc_sh.at[pl.ds(sid * RPS, RPS)],
                    out_hbm.at[pl.ds(cid * NPAD + sid * RPS, RPS)])


@functools.partial(
    pl.kernel,
    out_type=jax.ShapeDtypeStruct((NC * NPAD, D_HID), jnp.float32),
    mesh=_mesh,
    scratch_types=[
        pltpu.VMEM((NCHUNK, CHUNK), jnp.int32),
        pltpu.VMEM((NCHUNK, CHUNK), jnp.int32),
        pltpu.VMEM((CHUNK, D_HID), jnp.float32),
        pltpu.VMEM((CHUNK, D_HID), jnp.float32),
        pltpu.VMEM((CHUNK, D_HID), jnp.float32),
        pltpu.VMEM((CHUNK, D_HID), jnp.float32),
        pltpu.VMEM((CHUNK, D_HID), jnp.float32),
        pltpu.VMEM_SHARED((NPAD, D_HID), jnp.float32),
        pltpu.VMEM_SHARED((NPAD, D_HID), jnp.float32),
        pltpu.SemaphoreType.DMA,
        pltpu.SemaphoreType.DMA,
        pltpu.SemaphoreType.DMA,
        pltpu.SemaphoreType.DMA,
        pltpu.SemaphoreType.DMA,
        pltpu.SemaphoreType.DMA,
        pltpu.SemaphoreType.DMA,
        pltpu.SemaphoreType.DMA,
    ],
    compiler_params=pltpu.CompilerParams(use_tc_tiling_on_sc=False),
)
def _sc_aggregate(ht_hbm, row_hbm, col_hbm, out_hbm,
                  row_v, col_v, msg0_v, msg1_v, msg2_v, msg3_v, zeros_v,
                  acc_sh, ht_sh,
                  gsem0, gsem1, gsem2, gsem3, ssem0, ssem1, ssem2, ssem3):
    cid = lax.axis_index("c")
    sid = lax.axis_index("s")
    wid = sid * NC + cid
    _fill(zeros_v, CHUNK, 0.0)
    dh = pltpu.async_copy(ht_hbm.at[pl.ds(sid * RPS, RPS)],
                          ht_sh.at[pl.ds(sid * RPS, RPS)], gsem0)
    dr = pltpu.async_copy(row_hbm.at[pl.ds(wid * NCHUNK, NCHUNK)], row_v,
                          gsem1)
    dc = pltpu.async_copy(col_hbm.at[pl.ds(wid * NCHUNK, NCHUNK)], col_v,
                          gsem2)
    dz = [pltpu.async_copy(
        zeros_v, acc_sh.at[pl.ds(sid * RPS + k * CHUNK, CHUNK)], ssem0)
        for k in range(RPS // CHUNK)]
    dh.wait()
    dr.wait()
    dc.wait()
    for d in dz:
        d.wait()
    plsc.subcore_barrier()

    msgs = (msg0_v, msg1_v, msg2_v, msg3_v)
    gsems = (gsem0, gsem1, gsem2, gsem3)
    ssems = (ssem0, ssem1, ssem2, ssem3)

    def body(t, carry):
        gs = [pltpu.async_copy(ht_sh.at[row_v.at[4 * t + k]], msgs[k],
                               gsems[k]) for k in range(4)]
        ss = []
        for k in range(4):
            gs[k].wait()
            ss.append(pltpu.async_copy(
                msgs[k], acc_sh.at[col_v.at[4 * t + k]], ssems[k], add=True))
        for s in ss:
            s.wait()
        return carry

    lax.fori_loop(0, NCHUNK // 4, body, 0)
    plsc.subcore_barrier()
    pltpu.sync_copy(acc_sh.at[pl.ds(sid * RPS, RPS)],
                    out_hbm.at[pl.ds(cid * NPAD + sid * RPS, RPS)])


def _tc_pre(deg_ref, x3_ref, w1_ref, dinvp_ref, ht1p_ref):
    degp = deg_ref[0:NP8] + deg_ref[NP8:NC * NP8] + 1.0
    dinvp = lax.rsqrt(degp)
    dinvp_ref[...] = dinvp
    xw = [jnp.dot(x3_ref[:, a, :], w1_ref[...],
                  preferred_element_type=jnp.float32) for a in range(8)]
    ht1p_ref[0:NN8] = dinvp[0:NN8] * jnp.concatenate(xw, axis=1)
    ht1p_ref[NN8:NP8] = jnp.zeros((NP8 - NN8, 128), jnp.float32)


def _tc_mid(acc_ref, ht1p_ref, dinvp_ref, b1p_ref, ht2p_ref):
    dinvp = dinvp_ref[...]
    s = acc_ref[0:NP8] + acc_ref[NP8:NC * NP8] + ht1p_ref[...]
    out1 = jnp.maximum(dinvp * s + b1p_ref[...], 0.0)
    ht2p_ref[...] = dinvp * out1


def _tc_final(acc_ref, ht2p_ref, dinvp_ref, w2_ref, b2_ref, out3_ref):
    gp = dinvp_ref[0:NP8] * (
        acc_ref[0:NP8] + acc_ref[NP8:NC * NP8] + ht2p_ref[...])
    for a in range(8):
        ga = gp[0:NN8, 16 * a:16 * (a + 1)]
        h = jnp.dot(ga, w2_ref[...], preferred_element_type=jnp.float32)
        h = h + b2_ref[...]
        m = jnp.max(h, axis=1, keepdims=True)
        lse = m + jnp.log(jnp.sum(jnp.exp(h - m), axis=1, keepdims=True))
        out3_ref[:, a, :] = h - lse


def kernel(x, edge_index, W1, b1, W2, b2):
    ei = edge_index.astype(jnp.int32)
    row = jnp.concatenate([ei[0], jnp.asarray(_PAD_ROW)]).reshape(
        NW * NCHUNK, CHUNK)
    col = jnp.concatenate([ei[1], jnp.asarray(_PAD_COL)]).reshape(
        NW * NCHUNK, CHUNK)
    x3 = x.reshape(NN8, 8, D_IN)

    deg2 = _sc_degree(col)

    dinvp, ht1p = pl.pallas_call(
        _tc_pre,
        out_shape=[
            jax.ShapeDtypeStruct((NP8, 128), jnp.float32),
            jax.ShapeDtypeStruct((NP8, 128), jnp.float32),
        ],
    )(deg2.reshape(NC * NP8, 128), x3, W1)

    acc1 = _sc_aggregate(ht1p.reshape(NPAD, D_HID), row, col)

    ht2p = pl.pallas_call(
        _tc_mid,
        out_shape=jax.ShapeDtypeStruct((NP8, 128), jnp.float32),
    )(acc1.reshape(NC * NP8, 128), ht1p, dinvp,
      jnp.tile(b1.reshape(1, D_HID), (1, 8)))

    acc2 = _sc_aggregate(ht2p.reshape(NPAD, D_HID), row, col)

    out3 = pl.pallas_call(
        _tc_final,
        out_shape=jax.ShapeDtypeStruct((NN8, 8, D_OUT), jnp.float32),
    )(acc2.reshape(NC * NP8, 128), ht2p, dinvp, W2, b2.reshape(1, D_OUT))

    return out3.reshape(N_NODES, D_OUT)


# speedup vs baseline: 86.6714x; 1.0062x over previous
import functools

import jax
import jax.numpy as jnp
import numpy as np
from jax import lax
from jax.experimental import pallas as pl
from jax.experimental.pallas import tpu as pltpu
from jax.experimental.pallas import tpu_sc as plsc

N_NODES = 10000
N_EDGES = 320000
D_IN = 128
D_HID = 16
D_OUT = 128

NC = 2
NS = 16
NW = NC * NS
CHUNK = 128
NCHUNK = 80
EPAD = NW * NCHUNK * CHUNK
NPAD = 10240
RPS = NPAD // NS
NP8 = NPAD // 8
NN8 = N_NODES // 8

_PAD_ROW = np.zeros(EPAD - N_EDGES, np.int32)
_PAD_COL = (N_NODES + np.arange(EPAD - N_EDGES) % (NPAD - N_NODES)).astype(
    np.int32)

_mesh = plsc.VectorSubcoreMesh(
    core_axis_name="c", subcore_axis_name="s", num_cores=NC, num_subcores=NS)


def _fill(ref, n_rows, value):
    val = jnp.full((D_HID,), value, jnp.float32)

    def body(i, carry):
        ref[i] = val
        return carry

    lax.fori_loop(0, n_rows, body, 0)


@functools.partial(
    pl.kernel,
    out_type=jax.ShapeDtypeStruct((NC * NPAD, D_HID), jnp.float32),
    mesh=_mesh,
    scratch_types=[
        pltpu.VMEM((NCHUNK, CHUNK), jnp.int32),
        pltpu.VMEM((CHUNK, D_HID), jnp.float32),
        pltpu.VMEM((CHUNK, D_HID), jnp.float32),
        pltpu.VMEM_SHARED((NPAD, D_HID), jnp.float32),
        pltpu.SemaphoreType.DMA,
        pltpu.SemaphoreType.DMA,
        pltpu.SemaphoreType.DMA,
        pltpu.SemaphoreType.DMA,
    ],
    compiler_params=pltpu.CompilerParams(use_tc_tiling_on_sc=False),
)
def _sc_degree(col_hbm, out_hbm, col_v, ones_v, zeros_v, acc_sh,
               sem0, sem1, sem2, sem3):
    cid = lax.axis_index("c")
    sid = lax.axis_index("s")
    wid = sid * NC + cid
    _fill(ones_v, CHUNK, 1.0)
    _fill(zeros_v, CHUNK, 0.0)
    dc = pltpu.async_copy(col_hbm.at[pl.ds(wid * NCHUNK, NCHUNK)], col_v, sem0)
    dz = [pltpu.async_copy(
        zeros_v, acc_sh.at[pl.ds(sid * RPS + k * CHUNK, CHUNK)], sem1)
        for k in range(RPS // CHUNK)]
    dc.wait()
    for d in dz:
        d.wait()
    plsc.subcore_barrier()

    def body(t, carry):
        s0 = pltpu.async_copy(ones_v, acc_sh.at[col_v.at[4 * t]], sem0,
                              add=True)
        s1 = pltpu.async_copy(ones_v, acc_sh.at[col_v.at[4 * t + 1]], sem1,
                              add=True)
        s2 = pltpu.async_copy(ones_v, acc_sh.at[col_v.at[4 * t + 2]], sem2,
                              add=True)
        s3 = pltpu.async_copy(ones_v, acc_sh.at[col_v.at[4 * t + 3]], sem3,
                              add=True)
        s0.wait()
        s1.wait()
        s2.wait()
        s3.wait()
        return carry

    lax.fori_loop(0, NCHUNK // 4, body, 0)
    plsc.subcore_barrier()
    pltpu.sync_copy(acc_sh.at[pl.ds(sid * RPS, RPS)],
                    out_hbm.at[pl.ds(cid * NPAD + sid * RPS, RPS)])


@functools.partial(
    pl.kernel,
    out_type=jax.ShapeDtypeStruct((NC * NPAD, D_HID), jnp.float32),
    mesh=_mesh,
    scratch_types=[
        pltpu.VMEM((NCHUNK, CHUNK), jnp.int32),
        pltpu.VMEM((NCHUNK, CHUNK), jnp.int32),
        pltpu.VMEM((CHUNK, D_HID), jnp.float32),
        pltpu.VMEM((CHUNK, D_HID), jnp.float32),
        pltpu.VMEM((CHUNK, D_HID), jnp.float32),
        pltpu.VMEM((CHUNK, D_HID), jnp.float32),
        pltpu.VMEM((CHUNK, D_HID), jnp.float32),
        pltpu.VMEM_SHARED((NPAD, D_HID), jnp.float32),
        pltpu.VMEM_SHARED((NPAD, D_HID), jnp.float32),
        pltpu.SemaphoreType.DMA,
        pltpu.SemaphoreType.DMA,
        pltpu.SemaphoreType.DMA,
        pltpu.SemaphoreType.DMA,
        pltpu.SemaphoreType.DMA,
        pltpu.SemaphoreType.DMA,
        pltpu.SemaphoreType.DMA,
        pltpu.SemaphoreType.DMA,
    ],
    compiler_params=pltpu.CompilerParams(use_tc_tiling_on_sc=False),
)
def _sc_aggregate(ht_hbm, row_hbm, col_hbm, out_hbm,
                  row_v, col_v, msg0_v, msg1_v, msg2_v, msg3_v, zeros_v,
                  acc_sh, ht_sh,
                  gsem0, gsem1, gsem2, gsem3, ssem0, ssem1, ssem2, ssem3):
    cid = lax.axis_index("c")
    sid = lax.axis_index("s")
    wid = sid * NC + cid
    _fill(zeros_v, CHUNK, 0.0)
    dh = pltpu.async_copy(ht_hbm.at[pl.ds(sid * RPS, RPS)],
                          ht_sh.at[pl.ds(sid * RPS, RPS)], gsem0)
    dr = pltpu.async_copy(row_hbm.at[pl.ds(wid * NCHUNK, NCHUNK)], row_v,
                          gsem1)
    dc = pltpu.async_copy(col_hbm.at[pl.ds(wid * NCHUNK, NCHUNK)], col_v,
                          gsem2)
    dz = [pltpu.async_copy(
        zeros_v, acc_sh.at[pl.ds(sid * RPS + k * CHUNK, CHUNK)], ssem0)
        for k in range(RPS // CHUNK)]
    dh.wait()
    dr.wait()
    dc.wait()
    for d in dz:
        d.wait()
    plsc.subcore_barrier()

    msgs = (msg0_v, msg1_v, msg2_v, msg3_v)
    gsems = (gsem0, gsem1, gsem2, gsem3)
    ssems = (ssem0, ssem1, ssem2, ssem3)

    def body(t, carry):
        gs = [pltpu.async_copy(ht_sh.at[row_v.at[4 * t + k]], msgs[k],
                               gsems[k]) for k in range(4)]
        ss = []
        for k in range(4):
            gs[k].wait()
            ss.append(pltpu.async_copy(
                msgs[k], acc_sh.at[col_v.at[4 * t + k]], ssems[k], add=True))
        for s in ss:
            s.wait()
        return carry

    lax.fori_loop(0, NCHUNK // 4, body, 0)
    plsc.subcore_barrier()
    pltpu.sync_copy(acc_sh.at[pl.ds(sid * RPS, RPS)],
                    out_hbm.at[pl.ds(cid * NPAD + sid * RPS, RPS)])


def _tc_pre(deg_ref, x3_ref, w1p_ref, dinvp_ref, ht1p_ref):
    degp = deg_ref[0:NP8] + deg_ref[NP8:NC * NP8] + 1.0
    dinvp = lax.rsqrt(degp)
    dinvp_ref[...] = dinvp
    xw = jnp.dot(x3_ref[:, 0, :], w1p_ref[0],
                 preferred_element_type=jnp.float32)
    for a in range(1, 8):
        xw = xw + jnp.dot(x3_ref[:, a, :], w1p_ref[a],
                          preferred_element_type=jnp.float32)
    ht1p_ref[0:NN8] = dinvp[0:NN8] * xw
    ht1p_ref[NN8:NP8] = jnp.zeros((NP8 - NN8, 128), jnp.float32)


def _tc_mid(acc_ref, ht1p_ref, dinvp_ref, b1p_ref, ht2p_ref):
    dinvp = dinvp_ref[...]
    s = acc_ref[0:NP8] + acc_ref[NP8:NC * NP8] + ht1p_ref[...]
    out1 = jnp.maximum(dinvp * s + b1p_ref[...], 0.0)
    ht2p_ref[...] = dinvp * out1


def _tc_final(acc_ref, ht2p_ref, dinvp_ref, w2p_ref, b2_ref, out3_ref):
    gp = dinvp_ref[0:NP8] * (
        acc_ref[0:NP8] + acc_ref[NP8:NC * NP8] + ht2p_ref[...])
    gp = gp[0:NN8]
    for a in range(8):
        h = jnp.dot(gp, w2p_ref[a], preferred_element_type=jnp.float32)
        h = h + b2_ref[...]
        m = jnp.max(h, axis=1, keepdims=True)
        lse = m + jnp.log(jnp.sum(jnp.exp(h - m), axis=1, keepdims=True))
        out3_ref[:, a, :] = h - lse


def kernel(x, edge_index, W1, b1, W2, b2):
    ei = edge_index.astype(jnp.int32)
    row = jnp.concatenate([ei[0], jnp.asarray(_PAD_ROW)]).reshape(
        NW * NCHUNK, CHUNK)
    col = jnp.concatenate([ei[1], jnp.asarray(_PAD_COL)]).reshape(
        NW * NCHUNK, CHUNK)
    x3 = x.reshape(NN8, 8, D_IN)

    w1p = jnp.stack(
        [jnp.pad(W1, ((0, 0), (16 * a, 112 - 16 * a))) for a in range(8)])
    w2p = jnp.stack(
        [jnp.pad(W2, ((16 * a, 112 - 16 * a), (0, 0))) for a in range(8)])

    deg2 = _sc_degree(col)

    dinvp, ht1p = pl.pallas_call(
        _tc_pre,
        out_shape=[
            jax.ShapeDtypeStruct((NP8, 128), jnp.float32),
            jax.ShapeDtypeStruct((NP8, 128), jnp.float32),
        ],
    )(deg2.reshape(NC * NP8, 128), x3, w1p)

    acc1 = _sc_aggregate(ht1p.reshape(NPAD, D_HID), row, col)

    ht2p = pl.pallas_call(
        _tc_mid,
        out_shape=jax.ShapeDtypeStruct((NP8, 128), jnp.float32),
    )(acc1.reshape(NC * NP8, 128), ht1p, dinvp,
      jnp.tile(b1.reshape(1, D_HID), (1, 8)))

    acc2 = _sc_aggregate(ht2p.reshape(NPAD, D_HID), row, col)

    out3 = pl.pallas_call(
        _tc_final,
        out_shape=jax.ShapeDtypeStruct((NN8, 8, D_OUT), jnp.float32),
    )(acc2.reshape(NC * NP8, 128), ht2p, dinvp, w2p, b2.reshape(1, D_OUT))

    return out3.reshape(N_NODES, D_OUT)
